# baseline (speedup 1.0000x reference)
import jax
import jax.numpy as jnp
from jax import lax
from jax.experimental import pallas as pl
from jax.experimental.pallas import tpu as pltpu, tpu_sc as plsc

V = 100000
F = 26
D = 16
B = 4096
G = 25
GRID = G * G
N_DMA = 6
FO_BASE = 5 * 128
FW_OFF = F * V

NC, NS = 2, 16
NW = NC * NS
B_PER_W = B // NW
CB = 8
N_SUB = B_PER_W // CB


def _ffm_body(idx_hbm, vals_hbm, tab_hbm, out_hbm,
              idx_v, vals_v, pv_v, rows_v, out_v, sem):
    wid = lax.axis_index("s") * NC + lax.axis_index("c")
    base = wid * B_PER_W
    lane = lax.broadcasted_iota(jnp.int32, (D,), 0)

    def sub_chunk(c, _):
        b0 = base + c * CB
        pltpu.sync_copy(idx_hbm.at[pl.ds(b0, CB)], idx_v)
        pltpu.sync_copy(vals_hbm.at[pl.ds(b0, CB)], vals_v)

        def per_b(bb, res):
            copies = [
                pltpu.async_copy(
                    tab_hbm.at[idx_v.at[bb, g]],
                    rows_v.at[bb, pl.ds(g * 128, 128)],
                    sem,
                )
                for g in range(N_DMA)
            ]
            v0 = vals_v[bb, pl.ds(0, D)]
            v1 = vals_v[bb, pl.ds(D, D)]

            def pv_step(j, _):
                jj = jnp.zeros((D,), jnp.int32) + j
                lo = v0.at[jj].get(mode="promise_in_bounds")
                hi = v1.at[jj - D].get(mode="promise_in_bounds")
                pv_v[pl.ds(j * D, D)] = jnp.where(j < D, lo, hi)
                return 0

            lax.fori_loop(0, F, pv_step, 0)
            for cp in copies:
                cp.wait()

            def fo_step(f, acc):
                return acc + rows_v[bb, FO_BASE + f, :] * pv_v[pl.ds(f * D, D)]

            facc = lax.fori_loop(0, F, fo_step, jnp.zeros((D,), jnp.float32))

            def outer(i, acc):
                pvi = pv_v[pl.ds(i * D, D)]

                def inner(j, acc):
                    a = rows_v[bb, i * G + j, :]
                    b = rows_v[bb, j * G + i, :]
                    return acc + a * b * pvi * pv_v[pl.ds(j * D, D)]

                return lax.fori_loop(i + 1, G, inner, acc)

            acc = lax.fori_loop(0, G, outer, facc)
            for sh in (8, 4, 2, 1):
                acc = acc + acc.at[lane ^ sh].get(mode="promise_in_bounds")
            return jnp.where(lane == bb, acc, res)

        res = lax.fori_loop(0, CB, per_b, jnp.zeros((D,), jnp.float32))
        out_v[pl.ds(c * CB, D)] = res
        return 0

    lax.fori_loop(0, N_SUB, sub_chunk, 0)

    def sig(k, _):
        x = out_v[pl.ds(k * D, D)]
        out_v[pl.ds(k * D, D)] = 1.0 / (1.0 + jnp.exp(-x))
        return 0

    lax.fori_loop(0, B_PER_W // D, sig, 0)
    pltpu.sync_copy(out_v.at[pl.ds(0, B_PER_W)], out_hbm.at[pl.ds(base, B_PER_W)])


@jax.jit
def _ffm_call(idx_all, vals32, tab_all):
    mesh = plsc.VectorSubcoreMesh(core_axis_name="c", subcore_axis_name="s")
    return pl.kernel(
        _ffm_body,
        out_type=jax.ShapeDtypeStruct((B,), jnp.float32),
        mesh=mesh,
        compiler_params=pltpu.CompilerParams(use_tc_tiling_on_sc=False),
        scratch_types=[
            pltpu.VMEM((CB, N_DMA, 128), jnp.int32),
            pltpu.VMEM((CB, 2 * D), jnp.float32),
            pltpu.VMEM((F * D,), jnp.float32),
            pltpu.VMEM((CB, N_DMA * 128, D), jnp.float32),
            pltpu.VMEM((B_PER_W + D,), jnp.float32),
            pltpu.SemaphoreType.DMA,
        ],
    )(idx_all, vals32, tab_all)


def kernel(idxs, vals, emb_tables, first_w):
    tab_all = jnp.concatenate(
        [emb_tables.reshape(F * V, D),
         jnp.pad(first_w, ((0, 0), (0, D - 1)))], axis=0)
    ii = (jnp.arange(G, dtype=jnp.int32) * V)[None, :, None]
    grid = (ii + idxs[:, None, :G]).reshape(B, GRID)
    grid = jnp.pad(grid, ((0, 0), (0, FO_BASE - GRID))).reshape(B, 5, 128)
    fo = jnp.pad(idxs + FW_OFF, ((0, 0), (0, 128 - F)))[:, None, :]
    idx_all = jnp.concatenate([grid, fo], axis=1)
    vals32 = jnp.pad(vals, ((0, 0), (0, 2 * D - F)))
    return _ffm_call(idx_all, vals32, tab_all)


# speedup vs baseline: 5.9253x; 5.9253x over previous
import jax
import jax.numpy as jnp
from jax import lax
from jax.experimental import pallas as pl
from jax.experimental.pallas import tpu as pltpu, tpu_sc as plsc

V = 100000
F = 26
D = 16
B = 4096
G = 25
GRID = G * G
GRID_PAD = 640
N_DMA = GRID_PAD // 128
FO_PAD = 32

NC, NS = 2, 16
NW = NC * NS
B_PER_W = B // NW
CB = 8
N_SUB = B_PER_W // CB


def _ffm_body(idx_hbm, foidx_hbm, vals_hbm, emb_hbm, fw_hbm, out_hbm,
              idx_v, foidx_v, vals_v, pv_v, rows_v, fo_rows_v, out_v, sem):
    wid = lax.axis_index("s") * NC + lax.axis_index("c")
    base = wid * B_PER_W
    lane = lax.broadcasted_iota(jnp.int32, (D,), 0)

    def sub_chunk(c, _):
        b0 = base + c * CB
        pltpu.sync_copy(idx_hbm.at[pl.ds(b0, CB)], idx_v)
        pltpu.sync_copy(foidx_hbm.at[pl.ds(b0, CB)], foidx_v)
        pltpu.sync_copy(vals_hbm.at[pl.ds(b0, CB)], vals_v)

        def per_b(bb, res):
            copies = [
                pltpu.async_copy(
                    emb_hbm.at[idx_v.at[bb, g]],
                    rows_v.at[bb, pl.ds(g * 128, 128)],
                    sem,
                )
                for g in range(N_DMA)
            ]
            copies.append(
                pltpu.async_copy(fw_hbm.at[foidx_v.at[bb]], fo_rows_v.at[bb], sem)
            )
            v0 = vals_v[bb, pl.ds(0, D)]
            v1 = vals_v[bb, pl.ds(D, D)]

            def pv_step(j, _):
                jj = jnp.zeros((D,), jnp.int32) + j
                lo = v0.at[jj].get(mode="promise_in_bounds")
                hi = v1.at[jj - D].get(mode="promise_in_bounds")
                pv_v[pl.ds(j * D, D)] = jnp.where(j < D, lo, hi)
                return 0

            lax.fori_loop(0, F, pv_step, 0)
            for cp in copies:
                cp.wait()

            def fo_step(j, acc):
                return acc + fo_rows_v[bb, j, :] * pv_v[pl.ds(j * D, D)]

            facc = lax.fori_loop(0, F, fo_step, jnp.zeros((D,), jnp.float32))

            def outer(i, acc):
                pvi = pv_v[pl.ds(i * D, D)]

                def inner(j, acc):
                    a = rows_v[bb, i * G + j, :]
                    b = rows_v[bb, j * G + i, :]
                    return acc + a * b * pvi * pv_v[pl.ds(j * D, D)]

                return lax.fori_loop(i + 1, G, inner, acc)

            acc = lax.fori_loop(0, G, outer, facc)
            for sh in (8, 4, 2, 1):
                acc = acc + acc.at[lane ^ sh].get(mode="promise_in_bounds")
            return jnp.where(lane == bb, acc, res)

        res = lax.fori_loop(0, CB, per_b, jnp.zeros((D,), jnp.float32))
        out_v[pl.ds(c * CB, D)] = res
        return 0

    lax.fori_loop(0, N_SUB, sub_chunk, 0)

    def sig(k, _):
        x = out_v[pl.ds(k * D, D)]
        out_v[pl.ds(k * D, D)] = 1.0 / (1.0 + jnp.exp(-x))
        return 0

    lax.fori_loop(0, B_PER_W // D, sig, 0)
    pltpu.sync_copy(out_v.at[pl.ds(0, B_PER_W)], out_hbm.at[pl.ds(base, B_PER_W)])


@jax.jit
def _ffm_call(idx_grid, fo_idx, vals32, emb_flat, fw_pad):
    mesh = plsc.VectorSubcoreMesh(core_axis_name="c", subcore_axis_name="s")
    return pl.kernel(
        _ffm_body,
        out_type=jax.ShapeDtypeStruct((B,), jnp.float32),
        mesh=mesh,
        compiler_params=pltpu.CompilerParams(use_tc_tiling_on_sc=False),
        scratch_types=[
            pltpu.VMEM((CB, N_DMA, 128), jnp.int32),
            pltpu.VMEM((CB, FO_PAD), jnp.int32),
            pltpu.VMEM((CB, 2 * D), jnp.float32),
            pltpu.VMEM((F * D,), jnp.float32),
            pltpu.VMEM((CB, GRID_PAD, D), jnp.float32),
            pltpu.VMEM((CB, FO_PAD, D), jnp.float32),
            pltpu.VMEM((B_PER_W + D,), jnp.float32),
            pltpu.SemaphoreType.DMA,
        ],
    )(idx_grid, fo_idx, vals32, emb_flat, fw_pad)


def kernel(idxs, vals, emb_tables, first_w):
    emb_flat = emb_tables.reshape(F * V, D)
    fw_pad = jnp.pad(first_w, ((0, 0), (0, D - 1)))
    ii = (jnp.arange(G, dtype=jnp.int32) * V)[None, :, None]
    grid = (ii + idxs[:, None, :G]).reshape(B, GRID)
    idx_grid = jnp.pad(grid, ((0, 0), (0, GRID_PAD - GRID))).reshape(B, N_DMA, 128)
    fo_idx = jnp.pad(idxs, ((0, 0), (0, FO_PAD - F)))
    vals32 = jnp.pad(vals, ((0, 0), (0, 2 * D - F)))
    return _ffm_call(idx_grid, fo_idx, vals32, emb_flat, fw_pad)


# speedup vs baseline: 6.1225x; 1.0333x over previous
import jax
import jax.numpy as jnp
from jax import lax
from jax.experimental import pallas as pl
from jax.experimental.pallas import tpu as pltpu, tpu_sc as plsc

V = 100000
F = 26
D = 16
B = 4096
G = 25

NC, NS = 2, 16
NW = NC * NS
B_PER_W = B // NW
CB = 8
N_SUB = B_PER_W // CB


def _ffm_body(idx_hbm, vals_hbm, emb_hbm, fw_hbm, out_hbm,
              idx_v, vals_v, pv_v, rows_v, fo_rows_v, out_v, sem):
    wid = lax.axis_index("s") * NC + lax.axis_index("c")
    base = wid * B_PER_W
    lane = lax.broadcasted_iota(jnp.int32, (D,), 0)

    def sub_chunk(c, _):
        b0 = base + c * CB
        pltpu.sync_copy(idx_hbm.at[pl.ds(b0, CB)], idx_v)
        pltpu.sync_copy(vals_hbm.at[pl.ds(b0, CB)], vals_v)

        def per_b(bb, res):
            idx_row = idx_v.at[bb]
            copies = [
                pltpu.async_copy(
                    emb_hbm.at[i].at[idx_row], rows_v.at[bb, i], sem
                )
                for i in range(G)
            ]
            copies.append(
                pltpu.async_copy(fw_hbm.at[idx_row], fo_rows_v.at[bb], sem)
            )
            v0 = vals_v[bb, pl.ds(0, D)]
            v1 = vals_v[bb, pl.ds(D, D)]

            def pv_step(j, _):
                jj = jnp.zeros((D,), jnp.int32) + j
                lo = v0.at[jj].get(mode="promise_in_bounds")
                hi = v1.at[jj - D].get(mode="promise_in_bounds")
                pv_v[pl.ds(j * D, D)] = jnp.where(j < D, lo, hi)
                return 0

            lax.fori_loop(0, F, pv_step, 0)
            for cp in copies:
                cp.wait()

            def fo_step(f, acc):
                return acc + fo_rows_v[bb, f, :] * pv_v[pl.ds(f * D, D)]

            facc = lax.fori_loop(0, F, fo_step, jnp.zeros((D,), jnp.float32))

            def outer(i, acc):
                pvi = pv_v[pl.ds(i * D, D)]

                def inner(j, acc):
                    a = rows_v[bb, i, j, :]
                    b = rows_v[bb, j, i, :]
                    return acc + a * b * pvi * pv_v[pl.ds(j * D, D)]

                return lax.fori_loop(i + 1, G, inner, acc)

            acc = lax.fori_loop(0, G, outer, facc)
            for sh in (8, 4, 2, 1):
                acc = acc + acc.at[lane ^ sh].get(mode="promise_in_bounds")
            return jnp.where(lane == bb, acc, res)

        res = lax.fori_loop(0, CB, per_b, jnp.zeros((D,), jnp.float32))
        out_v[pl.ds(c * CB, D)] = res
        return 0

    lax.fori_loop(0, N_SUB, sub_chunk, 0)

    def sig(k, _):
        x = out_v[pl.ds(k * D, D)]
        out_v[pl.ds(k * D, D)] = 1.0 / (1.0 + jnp.exp(-x))
        return 0

    lax.fori_loop(0, B_PER_W // D, sig, 0)
    pltpu.sync_copy(out_v.at[pl.ds(0, B_PER_W)], out_hbm.at[pl.ds(base, B_PER_W)])


@jax.jit
def _ffm_call(idx32, vals32, emb_tables, fw_pad):
    mesh = plsc.VectorSubcoreMesh(core_axis_name="c", subcore_axis_name="s")
    return pl.kernel(
        _ffm_body,
        out_type=jax.ShapeDtypeStruct((B,), jnp.float32),
        mesh=mesh,
        compiler_params=pltpu.CompilerParams(use_tc_tiling_on_sc=False),
        scratch_types=[
            pltpu.VMEM((CB, 2 * D), jnp.int32),
            pltpu.VMEM((CB, 2 * D), jnp.float32),
            pltpu.VMEM((F * D,), jnp.float32),
            pltpu.VMEM((CB, G, 2 * D, D), jnp.float32),
            pltpu.VMEM((CB, 2 * D, D), jnp.float32),
            pltpu.VMEM((B_PER_W + D,), jnp.float32),
            pltpu.SemaphoreType.DMA,
        ],
    )(idx32, vals32, emb_tables, fw_pad)


def kernel(idxs, vals, emb_tables, first_w):
    idx32 = jnp.pad(idxs, ((0, 0), (0, 2 * D - F)))
    vals32 = jnp.pad(vals, ((0, 0), (0, 2 * D - F)))
    fw_pad = jnp.pad(first_w, ((0, 0), (0, D - 1)))
    return _ffm_call(idx32, vals32, emb_tables, fw_pad)


# speedup vs baseline: 6.7814x; 1.1076x over previous
import jax
import jax.numpy as jnp
from jax import lax
from jax.experimental import pallas as pl
from jax.experimental.pallas import tpu as pltpu, tpu_sc as plsc

V = 100000
F = 26
D = 16
B = 4096
G = 25
NPAIR = G * (G - 1) // 2
IW = 32

NC, NS = 2, 16
NW = NC * NS
B_PER_W = B // NW


def _ffm_body(idx_hbm, vals_hbm, emb_hbm, fw_hbm, out_hbm,
              idx_v, vals_v, rows_v, fo_v, out_v, pair_s, vs_s, sem0, sem1):
    wid = lax.axis_index("s") * NC + lax.axis_index("c")
    base = wid * B_PER_W
    lane = lax.broadcasted_iota(jnp.int32, (D,), 0)

    pltpu.sync_copy(idx_hbm.at[pl.ds(base, B_PER_W)], idx_v)
    pltpu.sync_copy(vals_hbm.at[pl.ds(base, B_PER_W)], vals_v)

    def pt_outer(i, p):
        def pt_inner(j, p):
            pair_s[p] = i * IW + j
            return p + 1

        return lax.fori_loop(i + 1, G, pt_inner, p)

    lax.fori_loop(0, G, pt_outer, 0)

    def fire(b, bk, sem):
        idx_row = idx_v.at[b]
        for i in range(G):
            pltpu.async_copy(emb_hbm.at[i].at[idx_row], rows_v.at[bk, i], sem)
        pltpu.async_copy(fw_hbm.at[idx_row], fo_v.at[bk], sem)

    def drain(bk, sem):
        idx_row = idx_v.at[0]
        for i in range(G):
            pltpu.make_async_copy(
                emb_hbm.at[i].at[idx_row], rows_v.at[bk, i], sem
            ).wait()
        pltpu.make_async_copy(fw_hbm.at[idx_row], fo_v.at[bk], sem).wait()

    def compute(b, bk, res):
        v0 = vals_v[b, pl.ds(0, D)]
        v1 = vals_v[b, pl.ds(D, D)]
        for f in range(F):
            vs_s[f] = (v0 if f < D else v1)[f % D]
        facc = fo_v[bk, pl.ds(0, D)] * v0 + fo_v[bk, pl.ds(D, D)] * v1

        def pk(k, accs):
            a0, a1 = accs
            pa = pair_s[2 * k]
            i0 = pa // IW
            j0 = pa % IW
            a0 = a0 + (rows_v[bk, i0, j0, :] * rows_v[bk, j0, i0, :]
                       * (vs_s[i0] * vs_s[j0]))
            pb = pair_s[2 * k + 1]
            i1 = pb // IW
            j1 = pb % IW
            a1 = a1 + (rows_v[bk, i1, j1, :] * rows_v[bk, j1, i1, :]
                       * (vs_s[i1] * vs_s[j1]))
            return (a0, a1)

        acc0, acc1 = lax.fori_loop(
            0, NPAIR // 2, pk, (facc, jnp.zeros((D,), jnp.float32)))
        acc = acc0 + acc1
        for sh in (8, 4, 2, 1):
            acc = acc + acc.at[lane ^ sh].get(mode="promise_in_bounds")
        return jnp.where(lane == (b & (D - 1)), acc, res)

    fire(0, 0, sem0)

    def step(m, res):
        b0 = 2 * m
        fire(b0 + 1, 1, sem1)
        drain(0, sem0)
        res = compute(b0, 0, res)

        @pl.when(m < B_PER_W // 2 - 1)
        def _():
            fire(b0 + 2, 0, sem0)

        drain(1, sem1)
        res = compute(b0 + 1, 1, res)

        @pl.when((b0 + 1) & (D - 1) == D - 1)
        def _():
            out_v[pl.ds(b0 + 1 - (D - 1), D)] = res

        return jnp.where((b0 + 1) & (D - 1) == D - 1,
                         jnp.zeros((D,), jnp.float32), res)

    lax.fori_loop(0, B_PER_W // 2, step, jnp.zeros((D,), jnp.float32))

    def sig(k, _):
        x = out_v[pl.ds(k * D, D)]
        out_v[pl.ds(k * D, D)] = 1.0 / (1.0 + jnp.exp(-x))
        return 0

    lax.fori_loop(0, B_PER_W // D, sig, 0)
    pltpu.sync_copy(out_v, out_hbm.at[pl.ds(base, B_PER_W)])


@jax.jit
def _ffm_call(idx32, vals32, emb_tables, fw_flat):
    mesh = plsc.VectorSubcoreMesh(core_axis_name="c", subcore_axis_name="s")
    return pl.kernel(
        _ffm_body,
        out_type=jax.ShapeDtypeStruct((B,), jnp.float32),
        mesh=mesh,
        compiler_params=pltpu.CompilerParams(use_tc_tiling_on_sc=False),
        scratch_types=[
            pltpu.VMEM((B_PER_W, IW), jnp.int32),
            pltpu.VMEM((B_PER_W, IW), jnp.float32),
            pltpu.VMEM((2, G, IW, D), jnp.float32),
            pltpu.VMEM((2, IW), jnp.float32),
            pltpu.VMEM((B_PER_W,), jnp.float32),
            pltpu.SMEM((NPAIR,), jnp.int32),
            pltpu.SMEM((IW,), jnp.float32),
            pltpu.SemaphoreType.DMA,
            pltpu.SemaphoreType.DMA,
        ],
    )(idx32, vals32, emb_tables, fw_flat)


def kernel(idxs, vals, emb_tables, first_w):
    idx32 = jnp.pad(idxs, ((0, 0), (0, IW - F)))
    vals32 = jnp.pad(vals, ((0, 0), (0, IW - F)))
    return _ffm_call(idx32, vals32, emb_tables, first_w.reshape(V))


# speedup vs baseline: 7.0198x; 1.0352x over previous
import jax
import jax.numpy as jnp
from jax import lax
from jax.experimental import pallas as pl
from jax.experimental.pallas import tpu as pltpu, tpu_sc as plsc

V = 100000
F = 26
D = 16
B = 4096
G = 25
NPAIR = G * (G - 1) // 2
IW = 32

NC, NS = 2, 16
NW = NC * NS
B_PER_W = B // NW


def _ffm_body(idx_hbm, vals_hbm, emb_hbm, fw_hbm, out_hbm,
              idx_v, vals_v, rows_v, fo_v, out_v, pair_s, vs_s, sem0, sem1):
    wid = lax.axis_index("s") * NC + lax.axis_index("c")
    base = wid * B_PER_W
    lane = lax.broadcasted_iota(jnp.int32, (D,), 0)

    pltpu.sync_copy(idx_hbm.at[pl.ds(base, B_PER_W), pl.ds(0, IW)], idx_v)
    pltpu.sync_copy(vals_hbm.at[pl.ds(base, B_PER_W), pl.ds(0, IW)], vals_v)

    def pt_outer(i, p):
        def pt_inner(j, p):
            pair_s[p] = i * IW + j
            return p + 1

        return lax.fori_loop(i + 1, G, pt_inner, p)

    lax.fori_loop(0, G, pt_outer, 0)

    def fire(b, bk, sem):
        idx_row = idx_v.at[b]
        for i in range(G):
            pltpu.async_copy(emb_hbm.at[i].at[idx_row], rows_v.at[bk, i], sem)
        pltpu.async_copy(fw_hbm.at[idx_row], fo_v.at[bk], sem)

    def drain(bk, sem):
        idx_row = idx_v.at[0]
        for i in range(G):
            pltpu.make_async_copy(
                emb_hbm.at[i].at[idx_row], rows_v.at[bk, i], sem
            ).wait()
        pltpu.make_async_copy(fw_hbm.at[idx_row], fo_v.at[bk], sem).wait()

    def compute(b, bk, res):
        v0 = vals_v[b, pl.ds(0, D)]
        v1 = vals_v[b, pl.ds(D, D)]
        for f in range(F):
            vs_s[f] = (v0 if f < D else v1)[f % D]
        facc = fo_v[bk, pl.ds(0, D)] * v0 + fo_v[bk, pl.ds(D, D)] * v1

        def pk(k, accs):
            a0, a1 = accs
            pa = pair_s[2 * k]
            i0 = pa // IW
            j0 = pa % IW
            a0 = a0 + (rows_v[bk, i0, j0, :] * rows_v[bk, j0, i0, :]
                       * (vs_s[i0] * vs_s[j0]))
            pb = pair_s[2 * k + 1]
            i1 = pb // IW
            j1 = pb % IW
            a1 = a1 + (rows_v[bk, i1, j1, :] * rows_v[bk, j1, i1, :]
                       * (vs_s[i1] * vs_s[j1]))
            return (a0, a1)

        acc0, acc1 = lax.fori_loop(
            0, NPAIR // 2, pk, (facc, jnp.zeros((D,), jnp.float32)))
        acc = acc0 + acc1
        for sh in (8, 4, 2, 1):
            acc = acc + acc.at[lane ^ sh].get(mode="promise_in_bounds")
        return jnp.where(lane == (b & (D - 1)), acc, res)

    fire(0, 0, sem0)

    def step(m, res):
        b0 = 2 * m
        fire(b0 + 1, 1, sem1)
        drain(0, sem0)
        res = compute(b0, 0, res)

        @pl.when(m < B_PER_W // 2 - 1)
        def _():
            fire(b0 + 2, 0, sem0)

        drain(1, sem1)
        res = compute(b0 + 1, 1, res)

        @pl.when((b0 + 1) & (D - 1) == D - 1)
        def _():
            out_v[pl.ds(b0 + 1 - (D - 1), D)] = res

        return jnp.where((b0 + 1) & (D - 1) == D - 1,
                         jnp.zeros((D,), jnp.float32), res)

    lax.fori_loop(0, B_PER_W // 2, step, jnp.zeros((D,), jnp.float32))

    def sig(k, _):
        x = out_v[pl.ds(k * D, D)]
        out_v[pl.ds(k * D, D)] = 1.0 / (1.0 + jnp.exp(-x))
        return 0

    lax.fori_loop(0, B_PER_W // D, sig, 0)
    pltpu.sync_copy(out_v, out_hbm.at[pl.ds(base, B_PER_W)])


@jax.jit
def _ffm_call(idx32, vals32, emb_tables, fw_flat):
    mesh = plsc.VectorSubcoreMesh(core_axis_name="c", subcore_axis_name="s")
    return pl.kernel(
        _ffm_body,
        out_type=jax.ShapeDtypeStruct((B,), jnp.float32),
        mesh=mesh,
        compiler_params=pltpu.CompilerParams(use_tc_tiling_on_sc=False),
        scratch_types=[
            pltpu.VMEM((B_PER_W, IW), jnp.int32),
            pltpu.VMEM((B_PER_W, IW), jnp.float32),
            pltpu.VMEM((2, G, IW, D), jnp.float32),
            pltpu.VMEM((2, IW), jnp.float32),
            pltpu.VMEM((B_PER_W,), jnp.float32),
            pltpu.SMEM((NPAIR,), jnp.int32),
            pltpu.SMEM((IW,), jnp.float32),
            pltpu.SemaphoreType.DMA,
            pltpu.SemaphoreType.DMA,
        ],
    )(idx32, vals32, emb_tables, fw_flat)


def kernel(idxs, vals, emb_tables, first_w):
    idx32 = jnp.pad(idxs, ((0, 0), (0, 128 - F)))
    vals32 = jnp.pad(vals, ((0, 0), (0, 128 - F)))
    return _ffm_call(idx32, vals32, emb_tables, first_w.reshape(V))
